# baseline (speedup 1.0000x reference)
import functools

import jax
import jax.numpy as jnp
from jax import lax
from jax.experimental import pallas as pl
from jax.experimental.pallas import tpu as pltpu
from jax.experimental.pallas import tpu_sc as plsc

_N_UTT = 8000
_N_REL = 2000
_N = 10000
_E = 320000
_H = 128
_R = 6
_NB = 30
_VOCAB = 64

_BLK = 1000
_NBLK = _N // _BLK

_NC = 2
_NS = 16
_NW = _NC * _NS
_CH = 80
_NCHUNK = 128
_NBUF = 3
_NHALF = 2
_HCH = 64
_EPW = _NCHUNK * _CH
_E_PAD = _EPW * _NW
_ACC_ROWS = 10112
_RPS = _ACC_ROWS // _NS


def _kw_body(comp_ref, bases_ref, w_ref):
    w_ref[...] = jnp.dot(comp_ref[...], bases_ref[...],
                         preferred_element_type=jnp.float32)


def _kw(comp, bases_flat):
    return pl.pallas_call(
        _kw_body,
        out_shape=jax.ShapeDtypeStruct((_R, _H * _H), jnp.float32),
    )(comp, bases_flat)


def _kx_core(x, w_ref, root_ref, gwt_ref, gb_ref, bias_ref,
             xwg_ref, xr_ref):
    gwt = gwt_ref[...]
    for r in range(_R):
        xwr = jnp.dot(x, w_ref[r], preferred_element_type=jnp.float32)
        g = jax.nn.sigmoid(jnp.sum(xwr * gwt, axis=1, keepdims=True)
                           + gb_ref[0, 0])
        xwg_ref[r] = xwr * g
    xr_ref[...] = (jnp.dot(x, root_ref[...], preferred_element_type=jnp.float32)
                   + bias_ref[...])


def _kx1_body(utt_ref, rels_ref, reltab_ref, w_ref, root_ref, gwt_ref,
              gb_ref, bias_ref, xwg_ref, xr_ref):
    i = pl.program_id(0)
    rr = rels_ref[0, 0]
    oh = (rr[:, None] == lax.broadcasted_iota(jnp.int32, (_BLK, _VOCAB), 1)
          ).astype(jnp.float32)
    embblk = jnp.dot(oh, reltab_ref[...], preferred_element_type=jnp.float32)
    x = jnp.where(i < _N_UTT // _BLK, utt_ref[...], embblk)
    _kx_core(x, w_ref, root_ref, gwt_ref, gb_ref, bias_ref, xwg_ref, xr_ref)


def _kx2_body(p_ref, xr1_ref, w_ref, root_ref, gwt_ref, gb_ref, bias_ref,
              xwg_ref, xr_ref):
    x = jnp.maximum(p_ref[0] + p_ref[1] + xr1_ref[...], 0.0)
    _kx_core(x, w_ref, root_ref, gwt_ref, gb_ref, bias_ref, xwg_ref, xr_ref)


_KX_COMMON_IN = [
    pl.BlockSpec((_R, _H, _H), lambda i: (0, 0, 0)),
    pl.BlockSpec((_H, _H), lambda i: (0, 0)),
    pl.BlockSpec((1, _H), lambda i: (0, 0)),
    pl.BlockSpec((1, 1), lambda i: (0, 0)),
    pl.BlockSpec((1, _H), lambda i: (0, 0)),
]

_KX_OUT = [
    pl.BlockSpec((_R, _BLK, _H), lambda i: (0, i, 0)),
    pl.BlockSpec((_BLK, _H), lambda i: (i, 0)),
]

_KX_OUT_SHAPE = [
    jax.ShapeDtypeStruct((_R, _N, _H), jnp.float32),
    jax.ShapeDtypeStruct((_N, _H), jnp.float32),
]


def _kx1(utt, rels3, reltab, w, root, gwt, gb, bias):
    return pl.pallas_call(
        _kx1_body,
        grid=(_NBLK,),
        in_specs=[
            pl.BlockSpec((_BLK, _H),
                         lambda i: (jnp.minimum(i, _N_UTT // _BLK - 1), 0)),
            pl.BlockSpec((1, 1, _BLK),
                         lambda i: (jnp.maximum(i - _N_UTT // _BLK, 0), 0, 0)),
            pl.BlockSpec((_VOCAB, _H), lambda i: (0, 0)),
        ] + _KX_COMMON_IN,
        out_specs=_KX_OUT,
        out_shape=_KX_OUT_SHAPE,
    )(utt, rels3, reltab, w, root, gwt, gb, bias)


def _kx2(partial, xr1, w, root, gwt, gb, bias):
    return pl.pallas_call(
        _kx2_body,
        grid=(_NBLK,),
        in_specs=[
            pl.BlockSpec((2, _BLK, _H), lambda i: (0, i, 0)),
            pl.BlockSpec((_BLK, _H), lambda i: (i, 0)),
        ] + _KX_COMMON_IN,
        out_specs=_KX_OUT,
        out_shape=_KX_OUT_SHAPE,
    )(partial, xr1, w, root, gwt, gb, bias)


def _kfin_body(p_ref, xr_ref, out_ref):
    out_ref[...] = p_ref[0] + p_ref[1] + xr_ref[...]


def _kfin(partial, xr):
    return pl.pallas_call(
        _kfin_body,
        grid=(_NBLK,),
        in_specs=[
            pl.BlockSpec((2, _BLK, _H), lambda i: (0, i, 0)),
            pl.BlockSpec((_BLK, _H), lambda i: (i, 0)),
        ],
        out_specs=pl.BlockSpec((_BLK, _H), lambda i: (i, 0)),
        out_shape=jax.ShapeDtypeStruct((_N, _H), jnp.float32),
    )(partial, xr)


_sc_mesh = plsc.VectorSubcoreMesh(core_axis_name="c", subcore_axis_name="s")


@functools.partial(
    pl.kernel,
    out_type=jax.ShapeDtypeStruct((_NC, _ACC_ROWS, _H), jnp.float32),
    mesh=_sc_mesh,
    scratch_types=[
        pltpu.VMEM((_HCH, _CH), jnp.int32),
        pltpu.VMEM((_HCH, _CH), jnp.int32),
        [pltpu.VMEM((_CH, _H), jnp.float32) for _ in range(_NBUF)],
        pltpu.VMEM_SHARED((_ACC_ROWS, _H), jnp.float32),
        pltpu.SemaphoreType.DMA,
        pltpu.SemaphoreType.DMA,
    ],
)
def _sc_edge_pass(xwg_hbm, idxw_hbm, dst_hbm, out_hbm,
                  idxw_v, dst_v, rows, acc, sem_r, sem_s):
    c = lax.axis_index("c")
    s = lax.axis_index("s")
    wid = c * _NS + s

    def _z(e, carry):
        for j in range(_H // 16):
            rows[0][e, pl.ds(j * 16, 16)] = jnp.zeros((16,), jnp.float32)
        return carry
    lax.fori_loop(0, _CH, _z, 0)
    row0 = s * _RPS
    for k in range(_RPS // _CH):
        pltpu.sync_copy(rows[0], acc.at[pl.ds(row0 + k * _CH, _CH)])
    _tail = _RPS % _CH
    if _tail:
        pltpu.sync_copy(rows[0].at[pl.ds(0, _tail)],
                        acc.at[pl.ds(row0 + _RPS - _tail, _tail)])
    plsc.subcore_barrier()

    for half in range(_NHALF):
        pltpu.sync_copy(idxw_hbm.at[wid, pl.ds(half * _HCH, _HCH)], idxw_v)
        pltpu.sync_copy(dst_hbm.at[wid, pl.ds(half * _HCH, _HCH)], dst_v)

        hs = [None] * (_HCH + 1)
        for ch in range(_HCH + 1):
            if ch < _HCH:
                hs[ch] = pltpu.async_copy(
                    xwg_hbm.at[idxw_v.at[ch]], rows[ch % _NBUF], sem_r)
            if ch >= 1:
                hs[ch - 1].wait()
                pltpu.sync_copy(rows[(ch - 1) % _NBUF],
                                acc.at[dst_v.at[ch - 1]], add=True)

    plsc.subcore_barrier()
    for k in range(_RPS // _CH):
        pltpu.sync_copy(acc.at[pl.ds(row0 + k * _CH, _CH)],
                        out_hbm.at[c, pl.ds(row0 + k * _CH, _CH)])
    if _tail:
        pltpu.sync_copy(acc.at[pl.ds(row0 + _RPS - _tail, _tail)],
                        out_hbm.at[c, pl.ds(row0 + _RPS - _tail, _tail)])


def _pad_i32(a, fill):
    return jnp.concatenate([a, fill]).reshape(_NW, _NCHUNK, _CH)


def kernel(meeting_utterance_enc_hidden_states, adj_coos, edge_types, rels,
           meeting_lens, rel_table, bases1, comp1, root1, bias1, gate_w1,
           gate_b1, bases2, comp2, root2, bias2, gate_w2, gate_b2):
    utt = meeting_utterance_enc_hidden_states
    src = adj_coos[0].astype(jnp.int32)
    dst = adj_coos[1].astype(jnp.int32)
    et = edge_types.astype(jnp.int32)
    npad = _E_PAD - _E
    pad_iota = jnp.arange(npad, dtype=jnp.int32)
    idxw = _pad_i32(et * _N + src, pad_iota % (_R * _N))
    dstp = _pad_i32(dst, _N + pad_iota % (_ACC_ROWS - _N))
    rels3 = rels.astype(jnp.int32).reshape(2, 1, _BLK)

    def layer(kx, xargs, bases, comp, root, bias, gate_w, gate_b):
        w = _kw(comp, bases.reshape(_NB, _H * _H)).reshape(_R, _H, _H)
        xwg, xr = kx(*xargs, w, root, gate_w.reshape(1, _H),
                     gate_b.reshape(1, 1), bias.reshape(1, _H))
        partial = _sc_edge_pass(xwg.reshape(_R * _N, _H), idxw, dstp)
        return partial, xr

    p1, xr1 = layer(_kx1, (utt, rels3, rel_table),
                    bases1, comp1, root1, bias1, gate_w1, gate_b1)
    p2, xr2 = layer(_kx2, (p1, xr1),
                    bases2, comp2, root2, bias2, gate_w2, gate_b2)
    return _kfin(p2, xr2)


# speedup vs baseline: 1.1076x; 1.1076x over previous
import functools

import jax
import jax.numpy as jnp
from jax import lax
from jax.experimental import pallas as pl
from jax.experimental.pallas import tpu as pltpu
from jax.experimental.pallas import tpu_sc as plsc

_N_UTT = 8000
_N_REL = 2000
_N = 10000
_E = 320000
_H = 128
_R = 6
_NB = 30
_VOCAB = 64

_BLK = 1000
_NBLK = _N // _BLK

_NC = 2
_NS = 16
_NW = _NC * _NS
_CH = 128
_NCHUNK = 80
_NBUF = 2
_NHALF = 2
_HCH = 40
_EPW = _NCHUNK * _CH
_E_PAD = _EPW * _NW
_ACC_ROWS = 10112
_RPS = _ACC_ROWS // _NS


def _kw_body(comp_ref, bases_ref, w_ref):
    w_ref[...] = jnp.dot(comp_ref[...], bases_ref[...],
                         preferred_element_type=jnp.float32)


def _kw(comp, bases_flat):
    return pl.pallas_call(
        _kw_body,
        out_shape=jax.ShapeDtypeStruct((_R, _H * _H), jnp.float32),
    )(comp, bases_flat)


def _kx_core(x, w_ref, root_ref, gwt_ref, gb_ref, bias_ref,
             xwg_ref, xr_ref):
    gwt = gwt_ref[...]
    for r in range(_R):
        xwr = jnp.dot(x, w_ref[r], preferred_element_type=jnp.float32)
        g = jax.nn.sigmoid(jnp.sum(xwr * gwt, axis=1, keepdims=True)
                           + gb_ref[0, 0])
        xwg_ref[r] = xwr * g
    xr_ref[...] = (jnp.dot(x, root_ref[...], preferred_element_type=jnp.float32)
                   + bias_ref[...])


def _kx1_body(utt_ref, rels_ref, reltab_ref, w_ref, root_ref, gwt_ref,
              gb_ref, bias_ref, xwg_ref, xr_ref):
    i = pl.program_id(0)
    rr = rels_ref[0, 0]
    oh = (rr[:, None] == lax.broadcasted_iota(jnp.int32, (_BLK, _VOCAB), 1)
          ).astype(jnp.float32)
    embblk = jnp.dot(oh, reltab_ref[...], preferred_element_type=jnp.float32)
    x = jnp.where(i < _N_UTT // _BLK, utt_ref[...], embblk)
    _kx_core(x, w_ref, root_ref, gwt_ref, gb_ref, bias_ref, xwg_ref, xr_ref)


def _kx2_body(p_ref, xr1_ref, w_ref, root_ref, gwt_ref, gb_ref, bias_ref,
              xwg_ref, xr_ref):
    x = jnp.maximum(p_ref[0] + p_ref[1] + xr1_ref[...], 0.0)
    _kx_core(x, w_ref, root_ref, gwt_ref, gb_ref, bias_ref, xwg_ref, xr_ref)


_KX_COMMON_IN = [
    pl.BlockSpec((_R, _H, _H), lambda i: (0, 0, 0)),
    pl.BlockSpec((_H, _H), lambda i: (0, 0)),
    pl.BlockSpec((1, _H), lambda i: (0, 0)),
    pl.BlockSpec((1, 1), lambda i: (0, 0)),
    pl.BlockSpec((1, _H), lambda i: (0, 0)),
]

_KX_OUT = [
    pl.BlockSpec((_R, _BLK, _H), lambda i: (0, i, 0)),
    pl.BlockSpec((_BLK, _H), lambda i: (i, 0)),
]

_KX_OUT_SHAPE = [
    jax.ShapeDtypeStruct((_R, _N, _H), jnp.float32),
    jax.ShapeDtypeStruct((_N, _H), jnp.float32),
]


def _kx1(utt, rels3, reltab, w, root, gwt, gb, bias):
    return pl.pallas_call(
        _kx1_body,
        grid=(_NBLK,),
        in_specs=[
            pl.BlockSpec((_BLK, _H),
                         lambda i: (jnp.minimum(i, _N_UTT // _BLK - 1), 0)),
            pl.BlockSpec((1, 1, _BLK),
                         lambda i: (jnp.maximum(i - _N_UTT // _BLK, 0), 0, 0)),
            pl.BlockSpec((_VOCAB, _H), lambda i: (0, 0)),
        ] + _KX_COMMON_IN,
        out_specs=_KX_OUT,
        out_shape=_KX_OUT_SHAPE,
    )(utt, rels3, reltab, w, root, gwt, gb, bias)


def _kx2(partial, xr1, w, root, gwt, gb, bias):
    return pl.pallas_call(
        _kx2_body,
        grid=(_NBLK,),
        in_specs=[
            pl.BlockSpec((2, _BLK, _H), lambda i: (0, i, 0)),
            pl.BlockSpec((_BLK, _H), lambda i: (i, 0)),
        ] + _KX_COMMON_IN,
        out_specs=_KX_OUT,
        out_shape=_KX_OUT_SHAPE,
    )(partial, xr1, w, root, gwt, gb, bias)


def _kfin_body(p_ref, xr_ref, out_ref):
    out_ref[...] = p_ref[0] + p_ref[1] + xr_ref[...]


def _kfin(partial, xr):
    return pl.pallas_call(
        _kfin_body,
        grid=(_NBLK,),
        in_specs=[
            pl.BlockSpec((2, _BLK, _H), lambda i: (0, i, 0)),
            pl.BlockSpec((_BLK, _H), lambda i: (i, 0)),
        ],
        out_specs=pl.BlockSpec((_BLK, _H), lambda i: (i, 0)),
        out_shape=jax.ShapeDtypeStruct((_N, _H), jnp.float32),
    )(partial, xr)


_sc_mesh = plsc.VectorSubcoreMesh(core_axis_name="c", subcore_axis_name="s")


@functools.partial(
    pl.kernel,
    out_type=jax.ShapeDtypeStruct((_NC, _ACC_ROWS, _H), jnp.float32),
    mesh=_sc_mesh,
    scratch_types=[
        pltpu.VMEM((_HCH, _CH), jnp.int32),
        pltpu.VMEM((_HCH, _CH), jnp.int32),
        pltpu.VMEM((_HCH, _CH), jnp.int32),
        [pltpu.VMEM((_CH, _H), jnp.float32) for _ in range(_NBUF)],
        pltpu.VMEM_SHARED((_ACC_ROWS, _H), jnp.float32),
        pltpu.SemaphoreType.DMA,
        pltpu.SemaphoreType.DMA,
    ],
)
def _sc_edge_pass(xwg_hbm, adj_hbm, et_hbm, out_hbm,
                  idxw_v, et_v, dst_v, rows, acc, sem_r, sem_s):
    c = lax.axis_index("c")
    s = lax.axis_index("s")
    wid = c * _NS + s

    def _z(e, carry):
        for j in range(_H // 16):
            rows[0][e, pl.ds(j * 16, 16)] = jnp.zeros((16,), jnp.float32)
        return carry
    lax.fori_loop(0, _CH, _z, 0)
    row0 = s * _RPS
    for k in range(_RPS // _CH):
        pltpu.sync_copy(rows[0], acc.at[pl.ds(row0 + k * _CH, _CH)])
    _tail = _RPS % _CH
    if _tail:
        pltpu.sync_copy(rows[0].at[pl.ds(0, _tail)],
                        acc.at[pl.ds(row0 + _RPS - _tail, _tail)])
    plsc.subcore_barrier()

    for half in range(_NHALF):
        pltpu.sync_copy(adj_hbm.at[0, wid, pl.ds(half * _HCH, _HCH)], idxw_v)
        pltpu.sync_copy(et_hbm.at[wid, pl.ds(half * _HCH, _HCH)], et_v)
        pltpu.sync_copy(adj_hbm.at[1, wid, pl.ds(half * _HCH, _HCH)], dst_v)

        def _mkidx(ch, carry):
            for j in range(_CH // 16):
                sl = pl.ds(j * 16, 16)
                idxw_v[ch, sl] = et_v[ch, sl] * _N + idxw_v[ch, sl]
            return carry
        lax.fori_loop(0, _HCH, _mkidx, 0)

        hs = [None] * (_HCH + 1)
        for ch in range(_HCH + 1):
            if ch < _HCH:
                hs[ch] = pltpu.async_copy(
                    xwg_hbm.at[idxw_v.at[ch]], rows[ch % _NBUF], sem_r)
            if ch >= 1:
                hs[ch - 1].wait()
                pltpu.sync_copy(rows[(ch - 1) % _NBUF],
                                acc.at[dst_v.at[ch - 1]], add=True)

    plsc.subcore_barrier()
    for k in range(_RPS // _CH):
        pltpu.sync_copy(acc.at[pl.ds(row0 + k * _CH, _CH)],
                        out_hbm.at[c, pl.ds(row0 + k * _CH, _CH)])
    if _tail:
        pltpu.sync_copy(acc.at[pl.ds(row0 + _RPS - _tail, _tail)],
                        out_hbm.at[c, pl.ds(row0 + _RPS - _tail, _tail)])


def _pad_i32(a, fill):
    return jnp.concatenate([a, fill], axis=-1).reshape(
        a.shape[:-1] + (_NW, _NCHUNK, _CH))


def kernel(meeting_utterance_enc_hidden_states, adj_coos, edge_types, rels,
           meeting_lens, rel_table, bases1, comp1, root1, bias1, gate_w1,
           gate_b1, bases2, comp2, root2, bias2, gate_w2, gate_b2):
    utt = meeting_utterance_enc_hidden_states
    npad = _E_PAD - _E
    pad_iota = jnp.arange(npad, dtype=jnp.int32)
    adjp = _pad_i32(
        adj_coos.astype(jnp.int32),
        jnp.stack([pad_iota % _N,
                   _N + pad_iota % (_ACC_ROWS - _N)]))
    etp = _pad_i32(edge_types.astype(jnp.int32), pad_iota % _R)
    rels3 = rels.astype(jnp.int32).reshape(2, 1, _BLK)

    def layer(kx, xargs, bases, comp, root, bias, gate_w, gate_b):
        w = _kw(comp, bases.reshape(_NB, _H * _H)).reshape(_R, _H, _H)
        xwg, xr = kx(*xargs, w, root, gate_w.reshape(1, _H),
                     gate_b.reshape(1, 1), bias.reshape(1, _H))
        partial = _sc_edge_pass(xwg.reshape(_R * _N, _H), adjp, etp)
        return partial, xr

    p1, xr1 = layer(_kx1, (utt, rels3, rel_table),
                    bases1, comp1, root1, bias1, gate_w1, gate_b1)
    p2, xr2 = layer(_kx2, (p1, xr1),
                    bases2, comp2, root2, bias2, gate_w2, gate_b2)
    return _kfin(p2, xr2)
